# baseline (speedup 1.0000x reference)
import jax
import jax.numpy as jnp
from jax.experimental import pallas as pl
from jax.experimental.pallas import tpu as pltpu

_BLK = 4096


def _gate_kernel(x_ref, wt_ref, b_ref, sim_ref, t_ref, o_ref):
    proj = jnp.dot(x_ref[...], wt_ref[...], preferred_element_type=jnp.float32)
    proj = proj + b_ref[...]
    nsq = jnp.sum(proj * proj, axis=-1, keepdims=True)
    a = jnp.maximum(nsq, 1e-24)
    inv = jax.lax.rsqrt(a)
    inv = inv * (1.5 - 0.5 * a * inv * inv)
    sim = sim_ref[...]
    cnorm = jnp.sqrt(jnp.sum(sim * sim, axis=0, keepdims=True))
    simn = (sim / jnp.maximum(cnorm, 1e-12)) * jnp.exp(t_ref[0, 0])
    out = jnp.dot(proj, simn, preferred_element_type=jnp.float32) * inv
    o_ref[...] = out.T


def kernel(x, W, b, sim_matrix, temperature):
    tokens, model_dim = x.shape
    proj_dim, _ = W.shape
    num_experts = sim_matrix.shape[1]
    wt = W.T
    b2 = b.reshape(1, proj_dim)
    t2 = temperature.reshape(1, 1)
    packed = pl.pallas_call(
        _gate_kernel,
        grid=(tokens // _BLK,),
        in_specs=[
            pl.BlockSpec((_BLK, model_dim), lambda i: (i, 0)),
            pl.BlockSpec((model_dim, proj_dim), lambda i: (0, 0)),
            pl.BlockSpec((1, proj_dim), lambda i: (0, 0)),
            pl.BlockSpec((proj_dim, num_experts), lambda i: (0, 0)),
            pl.BlockSpec((1, 1), lambda i: (0, 0)),
        ],
        out_specs=pl.BlockSpec((num_experts, _BLK), lambda i: (0, i)),
        out_shape=jax.ShapeDtypeStruct((num_experts, tokens), jnp.float32),
        compiler_params=pltpu.CompilerParams(
            dimension_semantics=("arbitrary",),
        ),
    )(x, wt, b2, sim_matrix, t2)
    return packed.T


# speedup vs baseline: 1.0112x; 1.0112x over previous
import jax
import jax.numpy as jnp
from jax.experimental import pallas as pl
from jax.experimental.pallas import tpu as pltpu

_BLK = 4096


def _gate_kernel(x_ref, wt_ref, b_ref, sim_ref, t_ref, o_ref):
    xb = x_ref[...].astype(jnp.bfloat16)
    proj = jnp.dot(xb, wt_ref[...], preferred_element_type=jnp.float32)
    projb = (proj + b_ref[...]).astype(jnp.bfloat16)
    p32 = projb.astype(jnp.float32)
    nsq = jnp.sum(p32 * p32, axis=-1, keepdims=True)
    inv = 1.0 / jnp.maximum(jnp.sqrt(nsq), 1e-12)
    sim = sim_ref[...]
    cnorm = jnp.sqrt(jnp.sum(sim * sim, axis=0, keepdims=True))
    simn = (sim / jnp.maximum(cnorm, 1e-12)) * jnp.exp(t_ref[0, 0])
    simn = simn.astype(jnp.bfloat16)
    out = jnp.dot(projb, simn, preferred_element_type=jnp.float32) * inv
    o_ref[...] = out.T


def kernel(x, W, b, sim_matrix, temperature):
    tokens, model_dim = x.shape
    proj_dim, _ = W.shape
    num_experts = sim_matrix.shape[1]
    wt = W.T.astype(jnp.bfloat16)
    b2 = b.reshape(1, proj_dim)
    t2 = temperature.reshape(1, 1)
    packed = pl.pallas_call(
        _gate_kernel,
        grid=(tokens // _BLK,),
        in_specs=[
            pl.BlockSpec((_BLK, model_dim), lambda i: (i, 0)),
            pl.BlockSpec((model_dim, proj_dim), lambda i: (0, 0)),
            pl.BlockSpec((1, proj_dim), lambda i: (0, 0)),
            pl.BlockSpec((proj_dim, num_experts), lambda i: (0, 0)),
            pl.BlockSpec((1, 1), lambda i: (0, 0)),
        ],
        out_specs=pl.BlockSpec((num_experts, _BLK), lambda i: (0, i)),
        out_shape=jax.ShapeDtypeStruct((num_experts, tokens), jnp.float32),
        compiler_params=pltpu.CompilerParams(
            dimension_semantics=("arbitrary",),
        ),
    )(x, wt, b2, sim_matrix, t2)
    return packed.T
